# baseline (device time: 55338 ns/iter reference)
import jax
import jax.numpy as jnp
from jax import lax
from jax.experimental import pallas as pl
from jax.experimental.pallas import tpu as pltpu

N_DEV = 16
NSUB = 8

PERM = [0, 1, 5, 9, 13, 14, 10, 6, 2, 3, 7, 11, 15, 12, 8, 4]
INV = [PERM.index(l) for l in range(N_DEV)]
NXT_CW = [PERM[(INV[l] + 1) % N_DEV] for l in range(N_DEV)]
NXT_CCW = [PERM[(INV[l] - 1) % N_DEV] for l in range(N_DEV)]


def _lut(table, idx):
    acc = jnp.int32(0)
    for j, v in enumerate(table):
        acc = acc + jnp.where(idx == j, jnp.int32(v), jnp.int32(0))
    return acc


def kernel(x, w_mat):
    m, k_per = x.shape
    _, n = w_mat.shape
    m_per = m // N_DEV
    nh = n // 2
    nb = nh // NSUB

    def body(x_ref, w_ref, out_ref,
             send_cw, send_ccw, recv_cw, recv_ccw,
             send_sems, recv_sems):
        my = lax.axis_index("i")
        r = _lut(INV, my)
        nxt_cw = _lut(NXT_CW, my)
        nxt_ccw = _lut(NXT_CCW, my)

        barrier = pltpu.get_barrier_semaphore()
        for nbr in (nxt_cw, nxt_ccw):
            pl.semaphore_signal(
                barrier, inc=1,
                device_id=(nbr,), device_id_type=pl.DeviceIdType.MESH,
            )
        pl.semaphore_wait(barrier, 2)

        w = w_ref[...].astype(jnp.bfloat16)

        def partial_chunk(ring_chunk, col0):
            row = _lut(PERM, ring_chunk)
            xc = x_ref[pl.ds(row * m_per, m_per), :].astype(jnp.bfloat16)
            return jnp.dot(xc, w[:, col0:col0 + nh],
                           preferred_element_type=jnp.float32)

        def mk(dirn, s, b, target):
            sb, rb = (send_cw, recv_cw) if dirn == 0 else (send_ccw, recv_ccw)
            return pltpu.make_async_remote_copy(
                src_ref=sb.at[s, b],
                dst_ref=rb.at[s, b],
                send_sem=send_sems.at[s, b, dirn],
                recv_sem=recv_sems.at[s, b, dirn],
                device_id=(target,),
                device_id_type=pl.DeviceIdType.MESH,
            )

        for s in range(N_DEV - 1):
            c_cw = lax.rem(r + (2 * N_DEV - 1 - s), N_DEV)
            c_ccw = lax.rem(r + s + 1, N_DEV)
            part_cw = partial_chunk(c_cw, 0)
            part_ccw = partial_chunk(c_ccw, nh)

            for b in range(NSUB):
                cs = b * nb
                if s == 0:
                    val_cw = part_cw[:, cs:cs + nb]
                else:
                    mk(0, s - 1, b, nxt_cw).wait_recv()
                    val_cw = part_cw[:, cs:cs + nb] + \
                        recv_cw[s - 1, b].astype(jnp.float32)
                send_cw[s, b] = val_cw.astype(jnp.bfloat16)
                mk(0, s, b, nxt_cw).start()
                if s == 0:
                    val_ccw = part_ccw[:, cs:cs + nb]
                else:
                    mk(1, s - 1, b, nxt_ccw).wait_recv()
                    val_ccw = part_ccw[:, cs:cs + nb] + \
                        recv_ccw[s - 1, b].astype(jnp.float32)
                send_ccw[s, b] = val_ccw.astype(jnp.bfloat16)
                mk(1, s, b, nxt_ccw).start()

        part_cw = partial_chunk(r, 0)
        part_ccw = partial_chunk(r, nh)
        for b in range(NSUB):
            cs = b * nb
            mk(0, N_DEV - 2, b, nxt_cw).wait_recv()
            fin = part_cw[:, cs:cs + nb] + \
                recv_cw[N_DEV - 2, b].astype(jnp.float32)
            out_ref[:, cs:cs + nb] = fin * jax.nn.sigmoid(fin)
            mk(1, N_DEV - 2, b, nxt_ccw).wait_recv()
            fin2 = part_ccw[:, cs:cs + nb] + \
                recv_ccw[N_DEV - 2, b].astype(jnp.float32)
            out_ref[:, nh + cs:nh + cs + nb] = fin2 * jax.nn.sigmoid(fin2)

        for s in range(N_DEV - 1):
            for b in range(NSUB):
                mk(0, s, b, nxt_cw).wait_send()
                mk(1, s, b, nxt_ccw).wait_send()

    return pl.pallas_call(
        body,
        out_shape=jax.ShapeDtypeStruct((m_per, n), jnp.float32),
        in_specs=[
            pl.BlockSpec(memory_space=pltpu.VMEM),
            pl.BlockSpec(memory_space=pltpu.VMEM),
        ],
        out_specs=pl.BlockSpec(memory_space=pltpu.VMEM),
        scratch_shapes=[
            pltpu.VMEM((N_DEV - 1, NSUB, m_per, nb), jnp.bfloat16),
            pltpu.VMEM((N_DEV - 1, NSUB, m_per, nb), jnp.bfloat16),
            pltpu.VMEM((N_DEV - 1, NSUB, m_per, nb), jnp.bfloat16),
            pltpu.VMEM((N_DEV - 1, NSUB, m_per, nb), jnp.bfloat16),
            pltpu.SemaphoreType.DMA((N_DEV - 1, NSUB, 2)),
            pltpu.SemaphoreType.DMA((N_DEV - 1, NSUB, 2)),
        ],
        compiler_params=pltpu.CompilerParams(collective_id=0),
    )(x, w_mat)


# device time: 52845 ns/iter; 1.0472x vs baseline; 1.0472x over previous
import jax
import jax.numpy as jnp
from jax import lax
from jax.experimental import pallas as pl
from jax.experimental.pallas import tpu as pltpu

N_DEV = 16
NSUB = 2
T_LONG = 8
T_SHORT = 7

PERM = [0, 1, 5, 9, 13, 14, 10, 6, 2, 3, 7, 11, 15, 12, 8, 4]
INV = [PERM.index(l) for l in range(N_DEV)]
NXT_CW = [PERM[(INV[l] + 1) % N_DEV] for l in range(N_DEV)]
NXT_CCW = [PERM[(INV[l] - 1) % N_DEV] for l in range(N_DEV)]


def _lut(table, idx):
    acc = jnp.int32(0)
    for j, v in enumerate(table):
        acc = acc + jnp.where(idx == j, jnp.int32(v), jnp.int32(0))
    return acc


def kernel(x, w_mat):
    m, k_per = x.shape
    _, n = w_mat.shape
    m_per = m // N_DEV
    nh = n // 2
    nb = nh // NSUB

    def body(x_ref, w_ref, out_ref, send_buf, recv_buf, send_sems, recv_sems):
        my = lax.axis_index("i")
        r = _lut(INV, my)
        rgt = _lut(NXT_CW, my)
        lft = _lut(NXT_CCW, my)

        fams = (
            (0, T_LONG, lft, lambda t: r + 8 + t),
            (0, T_SHORT, rgt, lambda t: r + 7 - t),
            (nh, T_LONG, rgt, lambda t: r + 8 - t),
            (nh, T_SHORT, lft, lambda t: r + 9 + t),
        )

        barrier = pltpu.get_barrier_semaphore()
        for nbr in (rgt, lft):
            pl.semaphore_signal(
                barrier, inc=1,
                device_id=(nbr,), device_id_type=pl.DeviceIdType.MESH,
            )
        pl.semaphore_wait(barrier, 2)

        w = w_ref[...].astype(jnp.bfloat16)

        def partial_chunk(chunk_expr, col0):
            c = lax.rem(chunk_expr, N_DEV)
            row = _lut(PERM, c)
            xc = x_ref[pl.ds(row * m_per, m_per), :].astype(jnp.bfloat16)
            return jnp.dot(xc, w[:, col0:col0 + nh],
                           preferred_element_type=jnp.float32)

        def mk(f, t, b, target):
            return pltpu.make_async_remote_copy(
                src_ref=send_buf.at[f, t, b],
                dst_ref=recv_buf.at[f, t, b],
                send_sem=send_sems.at[f, t, b],
                recv_sem=recv_sems.at[f, t, b],
                device_id=(target,),
                device_id_type=pl.DeviceIdType.MESH,
            )

        for t in range(T_LONG):
            parts = []
            for f, (col0, n_steps, target, chunk) in enumerate(fams):
                parts.append(partial_chunk(chunk(t), col0)
                             if t < n_steps else None)
            for b in range(NSUB):
                cs = b * nb
                for f, (col0, n_steps, target, chunk) in enumerate(fams):
                    if t >= n_steps:
                        continue
                    val = parts[f][:, cs:cs + nb]
                    if t > 0:
                        mk(f, t - 1, b, target).wait_recv()
                        val = val + recv_buf[f, t - 1, b].astype(jnp.float32)
                    send_buf[f, t, b] = val.astype(jnp.bfloat16)
                    mk(f, t, b, target).start()

        ph0 = partial_chunk(r, 0)
        ph1 = partial_chunk(r, nh)
        for b in range(NSUB):
            cs = b * nb
            mk(0, T_LONG - 1, b, lft).wait_recv()
            mk(1, T_SHORT - 1, b, rgt).wait_recv()
            fin = (ph0[:, cs:cs + nb]
                   + recv_buf[0, T_LONG - 1, b].astype(jnp.float32)
                   + recv_buf[1, T_SHORT - 1, b].astype(jnp.float32))
            out_ref[:, cs:cs + nb] = fin * jax.nn.sigmoid(fin)
            mk(2, T_LONG - 1, b, rgt).wait_recv()
            mk(3, T_SHORT - 1, b, lft).wait_recv()
            fin2 = (ph1[:, cs:cs + nb]
                    + recv_buf[2, T_LONG - 1, b].astype(jnp.float32)
                    + recv_buf[3, T_SHORT - 1, b].astype(jnp.float32))
            out_ref[:, nh + cs:nh + cs + nb] = fin2 * jax.nn.sigmoid(fin2)

        for f, (col0, n_steps, target, chunk) in enumerate(fams):
            for t in range(n_steps):
                for b in range(NSUB):
                    mk(f, t, b, target).wait_send()

    return pl.pallas_call(
        body,
        out_shape=jax.ShapeDtypeStruct((m_per, n), jnp.float32),
        in_specs=[
            pl.BlockSpec(memory_space=pltpu.VMEM),
            pl.BlockSpec(memory_space=pltpu.VMEM),
        ],
        out_specs=pl.BlockSpec(memory_space=pltpu.VMEM),
        scratch_shapes=[
            pltpu.VMEM((4, T_LONG, NSUB, m_per, nb), jnp.bfloat16),
            pltpu.VMEM((4, T_LONG, NSUB, m_per, nb), jnp.bfloat16),
            pltpu.SemaphoreType.DMA((4, T_LONG, NSUB)),
            pltpu.SemaphoreType.DMA((4, T_LONG, NSUB)),
        ],
        compiler_params=pltpu.CompilerParams(collective_id=0),
    )(x, w_mat)
